# baseline (speedup 1.0000x reference)
import functools

import jax
import jax.numpy as jnp
from jax import lax
from jax.experimental import pallas as pl
from jax.experimental.pallas import tpu as pltpu
from jax.experimental.pallas import tpu_sc as plsc

B = 16384
L = 20
D = 32
V_TITLE = 100001
VPAD = 100008
NC = 2
NS = 16
NW = NC * NS
BPW = B // NW
CH = 64
NCH = BPW // CH
TPW = 4

_SC_PARAMS = pltpu.CompilerParams(use_tc_tiling_on_sc=False,
                                  needs_layout_passes=False)


def _mesh():
    return plsc.VectorSubcoreMesh(core_axis_name="c", subcore_axis_name="s")


def _sc_detile_body(titleT, flat_out, row_v):
    wid = lax.axis_index("s") * NC + lax.axis_index("c")
    pltpu.sync_copy(titleT.at[wid], row_v)
    pltpu.sync_copy(row_v, flat_out.at[pl.ds(wid * VPAD, V_TITLE)])


def _sc_detile(titleT):
    kern = functools.partial(
        pl.kernel,
        mesh=_mesh(),
        out_type=jax.ShapeDtypeStruct((D * VPAD,), jnp.float32),
        scratch_types=[pltpu.VMEM((V_TITLE,), jnp.float32)],
        compiler_params=pltpu.CompilerParams(use_tc_tiling_on_sc=True,
                                             needs_layout_passes=False),
    )(_sc_detile_body)
    return kern(titleT)


def _fire_chunk(table_text, tt_v, g_v, sem, j):
    for l in range(L):
        pltpu.async_copy(table_text.at[tt_v.at[l, pl.ds(j * CH, CH)]],
                         g_v.at[pl.ds(l * CH, CH)], sem)


def _drain_chunk(table_text, g_v, sem):
    pltpu.make_async_copy(table_text.at[pl.ds(0, CH * L)], g_v, sem).wait()


def _sc_text_body(tokT, title2d, table_text, titleLin, e_sum_out, e_title_out,
                  tt_v, g0_v, g1_v, esT_v, tidx_v, etT_v, sem_a, sem_b, sem_t):
    wid = lax.axis_index("s") * NC + lax.axis_index("c")
    base = wid * BPW
    lane = lax.iota(jnp.int32, 16)

    pltpu.sync_copy(tokT.at[:, pl.ds(base, BPW)], tt_v)
    pltpu.sync_copy(title2d.at[pl.ds(wid * TPW, TPW)], tidx_v)

    def pool_chunk(j, g_v):
        def row_body(r, carry):
            a0 = g_v[r, 0:16]
            a1 = g_v[r, 16:32]
            b0 = g_v[CH + r, 0:16]
            b1 = g_v[CH + r, 16:32]
            for tkn in range(2, L, 2):
                a0 = a0 + g_v[tkn * CH + r, 0:16]
                a1 = a1 + g_v[tkn * CH + r, 16:32]
                b0 = b0 + g_v[(tkn + 1) * CH + r, 0:16]
                b1 = b1 + g_v[(tkn + 1) * CH + r, 16:32]
            col = jnp.full((16,), j * CH + r, jnp.int32)
            plsc.store_scatter(esT_v, [lane, col], a0 + b0)
            plsc.store_scatter(esT_v, [lane + 16, col], a1 + b1)
            return carry
        lax.fori_loop(0, CH, row_body, 0)

    _fire_chunk(table_text, tt_v, g0_v, sem_a, 0)
    _fire_chunk(table_text, tt_v, g1_v, sem_b, 1)

    def pair_body(g, carry):
        j0 = g * 2
        _drain_chunk(table_text, g0_v, sem_a)
        pool_chunk(j0, g0_v)

        @pl.when(j0 + 2 < NCH)
        def _():
            _fire_chunk(table_text, tt_v, g0_v, sem_a, j0 + 2)

        _drain_chunk(table_text, g1_v, sem_b)
        pool_chunk(j0 + 1, g1_v)

        @pl.when(j0 + 3 < NCH)
        def _():
            _fire_chunk(table_text, tt_v, g1_v, sem_b, j0 + 3)

        def title_fire(d, c2):
            for t in range(TPW):
                pltpu.async_copy(titleLin.at[d].at[tidx_v.at[t]],
                                 etT_v.at[d, pl.ds(t * 128, 128)], sem_t)
            return c2
        lax.fori_loop(g * (D // 4), (g + 1) * (D // 4), title_fire, 0)
        return carry

    lax.fori_loop(0, NCH // 2, pair_body, 0)
    pltpu.sync_copy(esT_v, e_sum_out.at[:, pl.ds(base, BPW)])
    pltpu.make_async_copy(titleLin.at[:, pl.ds(0, BPW)], etT_v, sem_t).wait()
    pltpu.sync_copy(etT_v, e_title_out.at[:, pl.ds(base, BPW)])


def _sc_text(tokT, title2d, table_text, titleLin):
    kern = functools.partial(
        pl.kernel,
        mesh=_mesh(),
        out_type=(jax.ShapeDtypeStruct((D, B), jnp.float32),
                  jax.ShapeDtypeStruct((D, B), jnp.float32)),
        scratch_types=[
            pltpu.VMEM((L, BPW), jnp.int32),
            pltpu.VMEM((CH * L, D), jnp.float32),
            pltpu.VMEM((CH * L, D), jnp.float32),
            pltpu.VMEM((D, BPW), jnp.float32),
            pltpu.VMEM((TPW, 128), jnp.int32),
            pltpu.VMEM((D, BPW), jnp.float32),
            pltpu.SemaphoreType.DMA,
            pltpu.SemaphoreType.DMA,
            pltpu.SemaphoreType.DMA,
        ],
        compiler_params=_SC_PARAMS,
    )(_sc_text_body)
    return kern(tokT, title2d, table_text, titleLin)


def _tc_mlp_body(tokT_ref, etT_ref, esT_ref, row0T_ref,
                 W1_ref, b1_ref, W2_ref, b2_ref, out_ref):
    nnz = jnp.sum((tokT_ref[...] != 0).astype(jnp.float32), axis=0,
                  keepdims=True)
    denom = jnp.maximum(nnz, 1.0)
    e_textT = (esT_ref[...] - row0T_ref[...] * (float(L) - nnz)) / denom
    W1 = W1_ref[...]
    hT = lax.dot_general(W1[:D], etT_ref[...], (((0,), (0,)), ((), ())),
                         preferred_element_type=jnp.float32)
    hT = hT + lax.dot_general(W1[D:], e_textT, (((0,), (0,)), ((), ())),
                              preferred_element_type=jnp.float32)
    hT = jnp.maximum(hT + b1_ref[...], 0.0)
    out_ref[...] = lax.dot_general(W2_ref[...], hT, (((0,), (0,)), ((), ())),
                                   preferred_element_type=jnp.float32) \
        + b2_ref[...]


def _tc_mlp(tokT, e_title_T, e_sum_T, row0T, W1, b1, W2, b2):
    blk = 2048
    grid = (B // blk,)
    return pl.pallas_call(
        _tc_mlp_body,
        grid=grid,
        in_specs=[
            pl.BlockSpec((L, blk), lambda i: (0, i)),
            pl.BlockSpec((D, blk), lambda i: (0, i)),
            pl.BlockSpec((D, blk), lambda i: (0, i)),
            pl.BlockSpec((D, 1), lambda i: (0, 0)),
            pl.BlockSpec((2 * D, 2 * D), lambda i: (0, 0)),
            pl.BlockSpec((2 * D, 1), lambda i: (0, 0)),
            pl.BlockSpec((2 * D, D), lambda i: (0, 0)),
            pl.BlockSpec((D, 1), lambda i: (0, 0)),
        ],
        out_specs=pl.BlockSpec((D, blk), lambda i: (0, i)),
        out_shape=jax.ShapeDtypeStruct((D, B), jnp.float32),
    )(tokT, e_title_T, e_sum_T, row0T, W1, b1, W2, b2)


def kernel(title_ids, token_ids, table_title, table_text, W1, b1, W2, b2):
    title2d = title_ids.reshape(B // 128, 128)
    tokT = token_ids.T
    titleLin = _sc_detile(table_title.T).reshape(D, VPAD)
    e_sum_T, e_title_T = _sc_text(tokT, title2d, table_text, titleLin)
    row0T = table_text[0].reshape(D, 1)
    outT = _tc_mlp(tokT, e_title_T, e_sum_T, row0T,
                   W1, b1.reshape(-1, 1), W2, b2.reshape(-1, 1))
    return outT.T


# speedup vs baseline: 1.0406x; 1.0406x over previous
import functools

import jax
import jax.numpy as jnp
from jax import lax
from jax.experimental import pallas as pl
from jax.experimental.pallas import tpu as pltpu
from jax.experimental.pallas import tpu_sc as plsc

B = 16384
L = 20
D = 32
V_TITLE = 100001
VPAD = 100008
NC = 2
NS = 16
NW = NC * NS
BPW = B // NW
CH = 64
NCH = BPW // CH
TPW = 4

_SC_PARAMS = pltpu.CompilerParams(use_tc_tiling_on_sc=False,
                                  needs_layout_passes=False)


def _mesh():
    return plsc.VectorSubcoreMesh(core_axis_name="c", subcore_axis_name="s")


def _sc_detile_body(titleT, flat_out, row_v):
    wid = lax.axis_index("s") * NC + lax.axis_index("c")
    pltpu.sync_copy(titleT.at[wid], row_v)
    pltpu.sync_copy(row_v, flat_out.at[pl.ds(wid * VPAD, V_TITLE)])


def _sc_detile(titleT):
    kern = functools.partial(
        pl.kernel,
        mesh=_mesh(),
        out_type=jax.ShapeDtypeStruct((D * VPAD,), jnp.float32),
        scratch_types=[pltpu.VMEM((V_TITLE,), jnp.float32)],
        compiler_params=pltpu.CompilerParams(use_tc_tiling_on_sc=True,
                                             needs_layout_passes=False),
    )(_sc_detile_body)
    return kern(titleT)


def _fire_chunk(table_text, tt_v, g_v, sem, j):
    for l in range(L):
        pltpu.async_copy(table_text.at[tt_v.at[l, pl.ds(j * CH, CH)]],
                         g_v.at[pl.ds(l * CH, CH)], sem)


def _drain_chunk(table_text, g_v, sem):
    pltpu.make_async_copy(table_text.at[pl.ds(0, CH * L)], g_v, sem).wait()


def _sc_text_body(tokT, title2d, table_text, titleLin, e_sum_out, e_title_out,
                  tt_v, g0_v, g1_v, esT_v, tidx_v, etT_v, sem_a, sem_b, sem_t):
    wid = lax.axis_index("s") * NC + lax.axis_index("c")
    base = wid * BPW
    lane = lax.iota(jnp.int32, 16)

    pltpu.sync_copy(tokT.at[:, pl.ds(base, BPW)], tt_v)
    pltpu.sync_copy(title2d.at[pl.ds(wid * TPW, TPW)], tidx_v)

    def pool_chunk(j, g_v):
        def row_body(r, carry):
            a0 = g_v[r, 0:16]
            a1 = g_v[r, 16:32]
            b0 = g_v[CH + r, 0:16]
            b1 = g_v[CH + r, 16:32]
            for tkn in range(2, L, 2):
                a0 = a0 + g_v[tkn * CH + r, 0:16]
                a1 = a1 + g_v[tkn * CH + r, 16:32]
                b0 = b0 + g_v[(tkn + 1) * CH + r, 0:16]
                b1 = b1 + g_v[(tkn + 1) * CH + r, 16:32]
            col = jnp.full((16,), j * CH + r, jnp.int32)
            plsc.store_scatter(esT_v, [lane, col], a0 + b0)
            plsc.store_scatter(esT_v, [lane + 16, col], a1 + b1)
            return carry
        lax.fori_loop(0, CH, row_body, 0)

    _fire_chunk(table_text, tt_v, g0_v, sem_a, 0)
    _fire_chunk(table_text, tt_v, g1_v, sem_b, 1)

    def pair_body(g, carry):
        j0 = g * 2
        _drain_chunk(table_text, g0_v, sem_a)
        pool_chunk(j0, g0_v)

        @pl.when(j0 + 2 < NCH)
        def _():
            _fire_chunk(table_text, tt_v, g0_v, sem_a, j0 + 2)

        _drain_chunk(table_text, g1_v, sem_b)
        pool_chunk(j0 + 1, g1_v)

        @pl.when(j0 + 3 < NCH)
        def _():
            _fire_chunk(table_text, tt_v, g1_v, sem_b, j0 + 3)

        def title_fire(d, c2):
            for t in range(TPW):
                pltpu.async_copy(titleLin.at[d].at[tidx_v.at[t]],
                                 etT_v.at[d, pl.ds(t * 128, 128)], sem_t)
            return c2
        lax.fori_loop(g * (D // 4), (g + 1) * (D // 4), title_fire, 0)
        return carry

    lax.fori_loop(0, NCH // 2, pair_body, 0)

    g = wid // 4
    for c in range(BPW // 128):
        pltpu.sync_copy(esT_v.at[:, pl.ds(c * 128, 128)],
                        e_sum_out.at[g, :, (wid % 4) * 4 + c])
    pltpu.make_async_copy(titleLin.at[:, pl.ds(0, BPW)], etT_v, sem_t).wait()
    for c in range(BPW // 128):
        pltpu.sync_copy(etT_v.at[:, pl.ds(c * 128, 128)],
                        e_title_out.at[g, :, (wid % 4) * 4 + c])


def _sc_text(tokT, title2d, table_text, titleLin):
    kern = functools.partial(
        pl.kernel,
        mesh=_mesh(),
        out_type=(jax.ShapeDtypeStruct((B // 2048, D, 16, 128), jnp.float32),
                  jax.ShapeDtypeStruct((B // 2048, D, 16, 128), jnp.float32)),
        scratch_types=[
            pltpu.VMEM((L, BPW), jnp.int32),
            pltpu.VMEM((CH * L, D), jnp.float32),
            pltpu.VMEM((CH * L, D), jnp.float32),
            pltpu.VMEM((D, BPW), jnp.float32),
            pltpu.VMEM((TPW, 128), jnp.int32),
            pltpu.VMEM((D, BPW), jnp.float32),
            pltpu.SemaphoreType.DMA,
            pltpu.SemaphoreType.DMA,
            pltpu.SemaphoreType.DMA,
        ],
        compiler_params=_SC_PARAMS,
    )(_sc_text_body)
    return kern(tokT, title2d, table_text, titleLin)


def _tc_mlp_body(tokT_ref, etT_ref, esT_ref, row0T_ref,
                 W1_ref, b1_ref, W2_ref, b2_ref, out_ref):
    blk = out_ref.shape[1]
    etT = etT_ref[...].reshape(D, blk)
    esT = esT_ref[...].reshape(D, blk)
    nnz = jnp.sum((tokT_ref[...] != 0).astype(jnp.float32), axis=0,
                  keepdims=True)
    denom = jnp.maximum(nnz, 1.0)
    e_textT = (esT - row0T_ref[...] * (float(L) - nnz)) / denom
    W1 = W1_ref[...]
    hT = lax.dot_general(W1[:D], etT, (((0,), (0,)), ((), ())),
                         preferred_element_type=jnp.float32)
    hT = hT + lax.dot_general(W1[D:], e_textT, (((0,), (0,)), ((), ())),
                              preferred_element_type=jnp.float32)
    hT = jnp.maximum(hT + b1_ref[...], 0.0)
    out_ref[...] = lax.dot_general(W2_ref[...], hT, (((0,), (0,)), ((), ())),
                                   preferred_element_type=jnp.float32) \
        + b2_ref[...]


def _tc_mlp(tokT, e_title_4d, e_sum_4d, row0T, W1, b1, W2, b2):
    blk = 2048
    grid = (B // blk,)
    return pl.pallas_call(
        _tc_mlp_body,
        grid=grid,
        in_specs=[
            pl.BlockSpec((L, blk), lambda i: (0, i)),
            pl.BlockSpec((1, D, 16, 128), lambda i: (i, 0, 0, 0)),
            pl.BlockSpec((1, D, 16, 128), lambda i: (i, 0, 0, 0)),
            pl.BlockSpec((D, 1), lambda i: (0, 0)),
            pl.BlockSpec((2 * D, 2 * D), lambda i: (0, 0)),
            pl.BlockSpec((2 * D, 1), lambda i: (0, 0)),
            pl.BlockSpec((2 * D, D), lambda i: (0, 0)),
            pl.BlockSpec((D, 1), lambda i: (0, 0)),
        ],
        out_specs=pl.BlockSpec((D, blk), lambda i: (0, i)),
        out_shape=jax.ShapeDtypeStruct((D, B), jnp.float32),
    )(tokT, e_title_4d, e_sum_4d, row0T, W1, b1, W2, b2)


def kernel(title_ids, token_ids, table_title, table_text, W1, b1, W2, b2):
    title2d = title_ids.reshape(B // 128, 128)
    tokT = token_ids.T
    titleLin = _sc_detile(table_title.T).reshape(D, VPAD)
    e_sum_T, e_title_T = _sc_text(tokT, title2d, table_text, titleLin)
    row0T = table_text[0].reshape(D, 1)
    outT = _tc_mlp(tokT, e_title_T, e_sum_T, row0T,
                   W1, b1.reshape(-1, 1), W2, b2.reshape(-1, 1))
    return outT.T
